# baseline (speedup 1.0000x reference)
import functools

import jax
import jax.numpy as jnp
from jax import lax
from jax.experimental import pallas as pl
from jax.experimental.pallas import tpu as pltpu
from jax.experimental.pallas import tpu_sc as plsc

_VTILE = 8192


def _mix_body(tT_ref, w_ref, o_ref):
    g = o_ref.shape[0]
    parts = [
        lax.dot_general(
            tT_ref[:, q * g:(q + 1) * g], w_ref[...],
            dimension_numbers=(((0,), (0,)), ((), ())),
            preferred_element_type=jnp.float32)
        for q in range(8)
    ]
    o_ref[...] = jnp.concatenate(parts, axis=1)


def _precompute_table(table_T, w):
    NE, V = table_T.shape
    E = w.shape[1]
    grid = (V + _VTILE - 1) // _VTILE
    return pl.pallas_call(
        _mix_body,
        grid=(grid,),
        in_specs=[pl.BlockSpec((NE, _VTILE), lambda i: (0, i)),
                  pl.BlockSpec((NE, E), lambda i: (0, 0))],
        out_specs=pl.BlockSpec((_VTILE // 8, 128), lambda i: (i, 0)),
        out_shape=jax.ShapeDtypeStruct((grid * (_VTILE // 8), 128),
                                       jnp.float32),
    )(table_T, w)


@functools.cache
def _make_sc_gather(V, Bb, Ss, E):
    info = plsc.get_sparse_core_info()
    NC, NS = info.num_cores, info.num_subcores
    NW = NC * NS
    b_per_w = Bb // NW
    CB = 32
    nch = b_per_w // CB
    ct = CB * Ss
    n_per_w = b_per_w * Ss
    mesh = plsc.VectorSubcoreMesh(core_axis_name="c", subcore_axis_name="s")

    TB = Bb // 128
    ET = E // 8

    @functools.partial(
        pl.kernel, mesh=mesh,
        out_type=jax.ShapeDtypeStruct((Ss, ET, TB, 8, 128), jnp.float32),
        scratch_types=[
            pltpu.VMEM((n_per_w,), jnp.int32),
            pltpu.VMEM((ct, E), jnp.float32),
            pltpu.VMEM((ct, E), jnp.float32),
            pltpu.VMEM((Ss, ET, 8, CB + 1), jnp.float32),
            pltpu.VMEM((Ss, ET, 8, CB + 1), jnp.float32),
            pltpu.SemaphoreType.DMA,
            pltpu.SemaphoreType.DMA,
        ],
        compiler_params=pltpu.CompilerParams(
            use_tc_tiling_on_sc=False, needs_layout_passes=False),
    )
    def k(table_hbm, idx_hbm, out_hbm, idx_v, rows_a, rows_b, trans_a,
          trans_b, sem_g, sem_o):
        wid = lax.axis_index("s") * NC + lax.axis_index("c")
        base = wid * n_per_w
        pltpu.sync_copy(idx_hbm.at[pl.ds(base, n_per_w)], idx_v)
        lane = lax.iota(jnp.int32, 16)
        tf = lane // 8
        fi = lane % 8

        VT = _VTILE
        G = VT // 8
        SG = G.bit_length() - 1

        def remap(i, _):
            vv = idx_v[pl.ds(i * 16, 16)]
            rem = vv & (VT - 1)
            idx_v[pl.ds(i * 16, 16)] = (
                (vv - rem) + ((rem & (G - 1)) << 3) + (rem >> SG))
            return 0

        lax.fori_loop(0, n_per_w // 16, remap, 0)

        bufs = [rows_a, rows_b]
        tbufs = [trans_a, trans_b]
        hg = [None] * nch
        ho = [None, None]
        hg[0] = pltpu.async_copy(
            table_hbm.at[idx_v.at[pl.ds(0, ct)]], bufs[0], sem_g)
        for c in range(nch):
            hg[c].wait()
            if c + 1 < nch:
                hg[c + 1] = pltpu.async_copy(
                    table_hbm.at[idx_v.at[pl.ds((c + 1) * ct, ct)]],
                    bufs[(c + 1) % 2], sem_g)
            if ho[c % 2] is not None:
                ho[c % 2].wait()
            rows_v = bufs[c % 2]
            trans_v = tbufs[c % 2]

            def body(bl, _):
                blv = jnp.full((16,), 0, jnp.int32) + bl
                K = 5
                for s0 in range(0, Ss, K):
                    grp = [rows_v[bl * Ss + s0 + j, :] for j in range(K)]
                    for j, row in enumerate(grp):
                        plsc.store_scatter(
                            trans_v,
                            [jnp.full((16,), s0 + j, jnp.int32), tf, fi,
                             blv], row)
                return 0

            lax.fori_loop(0, CB, body, 0)
            ho[c % 2] = pltpu.async_copy(
                trans_v.at[:, :, :, pl.ds(0, CB)],
                out_hbm.at[:, :, wid, :, pl.ds(c * CB, CB)],
                sem_o)
        for h in ho:
            if h is not None:
                h.wait()

    return k


def kernel(input_ids, state_embeddings, superposition_matrix):
    V, NSt, E = state_embeddings.shape
    Bb, Ss = input_ids.shape
    w = jnp.tile(superposition_matrix * (1.0 / NSt), (NSt, 1))
    table_T = state_embeddings.reshape(V, NSt * E).T
    t2 = _precompute_table(table_T, w)
    table2 = t2.reshape(t2.shape[0] * 8, E)
    flat_ids = input_ids.reshape(-1).astype(jnp.int32)
    out5 = _make_sc_gather(V, Bb, Ss, E)(table2, flat_ids)
    return jnp.transpose(out5, (2, 4, 0, 1, 3)).reshape(Bb, Ss, E)


# speedup vs baseline: 1.0153x; 1.0153x over previous
import functools

import jax
import jax.numpy as jnp
from jax import lax
from jax.experimental import pallas as pl
from jax.experimental.pallas import tpu as pltpu
from jax.experimental.pallas import tpu_sc as plsc

_VTILE = 8192


def _mix_body(tT_ref, w_ref, o_ref):
    g = o_ref.shape[0]
    parts = [
        lax.dot_general(
            tT_ref[:, q * g:(q + 1) * g], w_ref[...],
            dimension_numbers=(((0,), (0,)), ((), ())),
            preferred_element_type=jnp.float32)
        for q in range(8)
    ]
    o_ref[...] = jnp.concatenate(parts, axis=1)


def _precompute_table(table_T, w):
    NE, V = table_T.shape
    E = w.shape[1]
    grid = (V + _VTILE - 1) // _VTILE
    return pl.pallas_call(
        _mix_body,
        grid=(grid,),
        in_specs=[pl.BlockSpec((NE, _VTILE), lambda i: (0, i)),
                  pl.BlockSpec((NE, E), lambda i: (0, 0))],
        out_specs=pl.BlockSpec((_VTILE // 8, 128), lambda i: (i, 0)),
        out_shape=jax.ShapeDtypeStruct((grid * (_VTILE // 8), 128),
                                       jnp.float32),
    )(table_T, w)


@functools.cache
def _make_sc_gather(V, Bb, Ss, E):
    info = plsc.get_sparse_core_info()
    NC, NS = info.num_cores, info.num_subcores
    NW = NC * NS
    b_per_w = Bb // NW
    CB = 32
    nch = b_per_w // CB
    ct = CB * Ss
    n_per_w = b_per_w * Ss
    mesh = plsc.VectorSubcoreMesh(core_axis_name="c", subcore_axis_name="s")

    TB = Bb // 128
    ET = E // 8

    @functools.partial(
        pl.kernel, mesh=mesh,
        out_type=jax.ShapeDtypeStruct((Ss, ET, TB, 8, 128), jnp.float32),
        scratch_types=[
            pltpu.VMEM((n_per_w,), jnp.int32),
            pltpu.VMEM((ct, E), jnp.float32),
            pltpu.VMEM((ct, E), jnp.float32),
            pltpu.VMEM((Ss, ET, 8, CB + 1), jnp.float32),
            pltpu.VMEM((Ss, ET, 8, CB + 1), jnp.float32),
            pltpu.SemaphoreType.DMA,
            pltpu.SemaphoreType.DMA,
        ],
        compiler_params=pltpu.CompilerParams(
            use_tc_tiling_on_sc=False, needs_layout_passes=False),
    )
    def k(table_hbm, idx_hbm, out_hbm, idx_v, rows_a, rows_b, trans_a,
          trans_b, sem_g, sem_o):
        wid = lax.axis_index("s") * NC + lax.axis_index("c")
        base = wid * n_per_w
        pltpu.sync_copy(idx_hbm.at[pl.ds(base, n_per_w)], idx_v)
        lane = lax.iota(jnp.int32, 16)
        tf = lane // 8
        fi = lane % 8

        bufs = [rows_a, rows_b]
        tbufs = [trans_a, trans_b]
        hg = [None] * nch
        ho = [None, None]
        hg[0] = pltpu.async_copy(
            table_hbm.at[idx_v.at[pl.ds(0, ct)]], bufs[0], sem_g)
        for c in range(nch):
            hg[c].wait()
            if c + 1 < nch:
                hg[c + 1] = pltpu.async_copy(
                    table_hbm.at[idx_v.at[pl.ds((c + 1) * ct, ct)]],
                    bufs[(c + 1) % 2], sem_g)
            if ho[c % 2] is not None:
                ho[c % 2].wait()
            rows_v = bufs[c % 2]
            trans_v = tbufs[c % 2]

            def body(bl, _):
                blv = jnp.full((16,), 0, jnp.int32) + bl
                K = 5
                for s0 in range(0, Ss, K):
                    grp = [rows_v[bl * Ss + s0 + j, :] for j in range(K)]
                    for j, row in enumerate(grp):
                        plsc.store_scatter(
                            trans_v,
                            [jnp.full((16,), s0 + j, jnp.int32), tf, fi,
                             blv], row)
                return 0

            lax.fori_loop(0, CB, body, 0)
            ho[c % 2] = pltpu.async_copy(
                trans_v.at[:, :, :, pl.ds(0, CB)],
                out_hbm.at[:, :, wid, :, pl.ds(c * CB, CB)],
                sem_o)
        for h in ho:
            if h is not None:
                h.wait()

    return k


def kernel(input_ids, state_embeddings, superposition_matrix):
    V, NSt, E = state_embeddings.shape
    Bb, Ss = input_ids.shape
    w = jnp.tile(superposition_matrix * (1.0 / NSt), (NSt, 1))
    table_T = state_embeddings.reshape(V, NSt * E).T
    t2 = _precompute_table(table_T, w)
    table2 = t2.reshape(t2.shape[0] * 8, E)
    v = input_ids.reshape(-1).astype(jnp.int32)
    g = _VTILE // 8
    rem = v % _VTILE
    flat_ids = (v - rem) + (rem % g) * 8 + rem // g
    out5 = _make_sc_gather(V, Bb, Ss, E)(table2, flat_ids)
    return jnp.transpose(out5, (2, 4, 0, 1, 3)).reshape(Bb, Ss, E)
